# baseline (speedup 1.0000x reference)
import functools

import jax
import jax.numpy as jnp
from jax import lax
from jax.experimental import pallas as pl
from jax.experimental.pallas import tpu as pltpu
from jax.experimental.pallas import tpu_sc as plsc

NUM_RELATIONS = 1000
FUSE_DIM = 64
BATCH = 16384

_NC = 1
_NS = 16
_NW = _NC * _NS
_CHUNK = BATCH // _NW
_SUB = 512
_NSUB = _CHUNK // _SUB
_GROUPS = _SUB // 16


def _sc_kernel(h_hbm, idx_hbm, t_hbm, rel_hbm, out_hbm,
               idx_v, h_v, t_v, r_v, out_v, sem_h, sem_t, sem_r):
    wid = lax.axis_index("s") * _NC + lax.axis_index("c")
    base = wid * _CHUNK

    iota = lax.iota(jnp.int32, 16)

    for sub in range(_NSUB):
        sbase = base + sub * _SUB
        pltpu.sync_copy(idx_hbm.at[pl.ds(sbase, _SUB)], idx_v)
        cp_r = pltpu.async_copy(rel_hbm.at[idx_v], r_v, sem_r)
        cp_h = pltpu.async_copy(h_hbm.at[pl.ds(sbase, _SUB), :], h_v, sem_h)
        cp_t = pltpu.async_copy(t_hbm.at[pl.ds(sbase, _SUB), :], t_v, sem_t)
        cp_h.wait()
        cp_t.wait()
        cp_r.wait()

        def group_body(g, carry):
            rows = iota + g * 16

            def d_body(dd, acc):
                cols = lax.bitwise_and(iota + dd, FUSE_DIM - 1)
                hv = plsc.load_gather(h_v, [rows, cols])
                rv = plsc.load_gather(r_v, [rows, cols])
                tv = plsc.load_gather(t_v, [rows, cols])
                return acc + hv * rv * tv

            acc = lax.fori_loop(0, FUSE_DIM, d_body,
                                jnp.zeros((16,), jnp.float32), unroll=8)
            out_v[pl.ds(sub * _SUB + g * 16, 16)] = acc
            return carry

        lax.fori_loop(0, _GROUPS, group_body, 0)

    pltpu.sync_copy(out_v, out_hbm.at[pl.ds(base, _CHUNK)])


@jax.jit
def kernel(h_emb, r_idx, t_emb, rel_emb):
    mesh = plsc.VectorSubcoreMesh(
        core_axis_name="c", subcore_axis_name="s", num_cores=_NC)
    run = functools.partial(
        pl.kernel,
        mesh=mesh,
        compiler_params=pltpu.CompilerParams(
            needs_layout_passes=False, use_tc_tiling_on_sc=False),
        out_type=jax.ShapeDtypeStruct((BATCH,), jnp.float32),
        scratch_types=[
            pltpu.VMEM((_SUB,), jnp.int32),
            pltpu.VMEM((_SUB, FUSE_DIM), jnp.float32),
            pltpu.VMEM((_SUB, FUSE_DIM), jnp.float32),
            pltpu.VMEM((_SUB, FUSE_DIM), jnp.float32),
            pltpu.VMEM((_CHUNK,), jnp.float32),
            pltpu.SemaphoreType.DMA,
            pltpu.SemaphoreType.DMA,
            pltpu.SemaphoreType.DMA,
        ],
    )(_sc_kernel)
    return run(h_emb, r_idx.astype(jnp.int32), t_emb, rel_emb)


# speedup vs baseline: 1.3182x; 1.3182x over previous
import functools

import jax
import jax.numpy as jnp
from jax import lax
from jax.experimental import pallas as pl
from jax.experimental.pallas import tpu as pltpu
from jax.experimental.pallas import tpu_sc as plsc

NUM_RELATIONS = 1000
FUSE_DIM = 64
BATCH = 16384

_NC = 2
_NS = 16
_NW = _NC * _NS
_CHUNK = BATCH // _NW
_SUB = 256
_NSUB = _CHUNK // _SUB
_GROUPS = _SUB // 16


def _sc_kernel(h_hbm, idx_hbm, t_hbm, rel_hbm, out_hbm,
               idx_v, h_v, t_v, r_v, out_v, sem_h, sem_t, sem_r):
    wid = lax.axis_index("s") * _NC + lax.axis_index("c")
    base = wid * _CHUNK

    iota = lax.iota(jnp.int32, 16)

    for sub in range(_NSUB):
        sbase = pl.multiple_of(base + sub * _SUB, 8)
        pltpu.sync_copy(idx_hbm.at[pl.ds(sbase, _SUB)], idx_v)
        cp_r = pltpu.async_copy(rel_hbm.at[idx_v], r_v, sem_r)
        cp_h = pltpu.async_copy(h_hbm.at[pl.ds(sbase, _SUB), :], h_v, sem_h)
        cp_t = pltpu.async_copy(t_hbm.at[pl.ds(sbase, _SUB), :], t_v, sem_t)
        cp_h.wait()
        cp_t.wait()
        cp_r.wait()

        def group_body(g, carry):
            rows = iota + g * 16

            def d_body(dd, acc):
                cols = lax.bitwise_and(iota + dd, FUSE_DIM - 1)
                hv = plsc.load_gather(h_v, [rows, cols])
                rv = plsc.load_gather(r_v, [rows, cols])
                tv = plsc.load_gather(t_v, [rows, cols])
                return acc + hv * rv * tv

            acc = lax.fori_loop(0, FUSE_DIM, d_body,
                                jnp.zeros((16,), jnp.float32), unroll=8)
            out_v[pl.ds(sub * _SUB + g * 16, 16)] = acc
            return carry

        lax.fori_loop(0, _GROUPS, group_body, 0)

    pltpu.sync_copy(out_v, out_hbm.at[pl.ds(base, _CHUNK)])


@jax.jit
def kernel(h_emb, r_idx, t_emb, rel_emb):
    relp = jnp.pad(rel_emb, ((0, 0), (0, 128 - FUSE_DIM)))
    mesh = plsc.VectorSubcoreMesh(core_axis_name="c", subcore_axis_name="s")
    run = functools.partial(
        pl.kernel,
        mesh=mesh,
        compiler_params=pltpu.CompilerParams(needs_layout_passes=False),
        out_type=jax.ShapeDtypeStruct((BATCH,), jnp.float32),
        scratch_types=[
            pltpu.VMEM((_SUB,), jnp.int32),
            pltpu.VMEM((_SUB, FUSE_DIM), jnp.float32),
            pltpu.VMEM((_SUB, FUSE_DIM), jnp.float32),
            pltpu.VMEM((_SUB, 128), jnp.float32),
            pltpu.VMEM((_CHUNK,), jnp.float32),
            pltpu.SemaphoreType.DMA,
            pltpu.SemaphoreType.DMA,
            pltpu.SemaphoreType.DMA,
        ],
    )(_sc_kernel)
    return run(h_emb, r_idx.astype(jnp.int32), t_emb, relp)


# speedup vs baseline: 1.7966x; 1.3629x over previous
import functools

import jax
import jax.numpy as jnp
from jax import lax
from jax.experimental import pallas as pl
from jax.experimental.pallas import tpu as pltpu
from jax.experimental.pallas import tpu_sc as plsc

NUM_RELATIONS = 1000
FUSE_DIM = 64
BATCH = 16384

_NC = 2
_NS = 16
_NW = _NC * _NS
_CHUNK = BATCH // _NW
_SUB = 256
_NSUB = _CHUNK // _SUB
_GROUPS = _SUB // 16


def _sc_kernel(ht_hbm, idx_hbm, tt_hbm, rel_hbm, out_hbm,
               idx_v, h_v, t_v, r_v, out_v, sem_h, sem_t, sem_r):
    wid = lax.axis_index("s") * _NC + lax.axis_index("c")
    base = wid * _CHUNK

    iota = lax.iota(jnp.int32, 16)

    for sub in range(_NSUB):
        sbase = pl.multiple_of(base + sub * _SUB, _SUB)
        pltpu.sync_copy(idx_hbm.at[pl.ds(sbase, _SUB)], idx_v)
        cp_r = pltpu.async_copy(rel_hbm.at[idx_v], r_v, sem_r)
        cp_h = pltpu.async_copy(ht_hbm.at[:, pl.ds(sbase, _SUB)], h_v, sem_h)
        cp_t = pltpu.async_copy(tt_hbm.at[:, pl.ds(sbase, _SUB)], t_v, sem_t)
        cp_h.wait()
        cp_t.wait()
        cp_r.wait()

        def group_body(g, carry):
            rows = iota + g * 16

            def d_body(dd, acc):
                cols = lax.bitwise_and(iota + dd, FUSE_DIM - 1)
                hv = plsc.load_gather(h_v, [cols, rows])
                tv = plsc.load_gather(t_v, [cols, rows])
                rv = plsc.load_gather(r_v, [rows, cols])
                return acc + hv * rv * tv

            acc = lax.fori_loop(0, FUSE_DIM, d_body,
                                jnp.zeros((16,), jnp.float32), unroll=8)
            out_v[pl.ds(sub * _SUB + g * 16, 16)] = acc
            return carry

        lax.fori_loop(0, _GROUPS, group_body, 0)

    pltpu.sync_copy(out_v, out_hbm.at[pl.ds(base, _CHUNK)])


@jax.jit
def kernel(h_emb, r_idx, t_emb, rel_emb):
    ht = h_emb.T
    tt = t_emb.T
    relp = jnp.pad(rel_emb, ((0, 0), (0, 128 - FUSE_DIM)))
    mesh = plsc.VectorSubcoreMesh(core_axis_name="c", subcore_axis_name="s")
    run = functools.partial(
        pl.kernel,
        mesh=mesh,
        compiler_params=pltpu.CompilerParams(needs_layout_passes=False),
        out_type=jax.ShapeDtypeStruct((BATCH,), jnp.float32),
        scratch_types=[
            pltpu.VMEM((_SUB,), jnp.int32),
            pltpu.VMEM((FUSE_DIM, _SUB), jnp.float32),
            pltpu.VMEM((FUSE_DIM, _SUB), jnp.float32),
            pltpu.VMEM((_SUB, 128), jnp.float32),
            pltpu.VMEM((_CHUNK,), jnp.float32),
            pltpu.SemaphoreType.DMA,
            pltpu.SemaphoreType.DMA,
            pltpu.SemaphoreType.DMA,
        ],
    )(_sc_kernel)
    return run(ht, r_idx.astype(jnp.int32), tt, relp)


# speedup vs baseline: 1.8732x; 1.0427x over previous
import functools

import jax
import jax.numpy as jnp
from jax import lax
from jax.experimental import pallas as pl
from jax.experimental.pallas import tpu as pltpu
from jax.experimental.pallas import tpu_sc as plsc

NUM_RELATIONS = 1000
FUSE_DIM = 64
BATCH = 16384

_NC = 2
_NS = 16
_NW = _NC * _NS
_CHUNK = BATCH // _NW
_SUB = 128
_NSUB = _CHUNK // _SUB
_GROUPS = _SUB // 16


def _sc_kernel(ht_hbm, idx_hbm, tt_hbm, rel_hbm, out_hbm,
               idx_v0, h_v0, t_v0, r_v0, sh0, st0, sr0,
               idx_v1, h_v1, t_v1, r_v1, sh1, st1, sr1,
               out_v):
    bufs = ((idx_v0, h_v0, t_v0, r_v0, sh0, st0, sr0),
            (idx_v1, h_v1, t_v1, r_v1, sh1, st1, sr1))
    wid = lax.axis_index("s") * _NC + lax.axis_index("c")
    base = wid * _CHUNK

    iota = lax.iota(jnp.int32, 16)

    def start(sub):
        idx_v, h_v, t_v, r_v, sh, st, sr = bufs[sub % 2]
        sbase = pl.multiple_of(base + sub * _SUB, _SUB)
        pltpu.sync_copy(idx_hbm.at[pl.ds(sbase, _SUB)], idx_v)
        return (pltpu.async_copy(rel_hbm.at[idx_v], r_v, sr),
                pltpu.async_copy(ht_hbm.at[:, pl.ds(sbase, _SUB)], h_v, sh),
                pltpu.async_copy(tt_hbm.at[:, pl.ds(sbase, _SUB)], t_v, st))

    def compute(sub):
        _, h_v, t_v, r_v, *_ = bufs[sub % 2]

        def group_body(g, carry):
            rows = iota + g * 16

            def d_body(dd, acc):
                cols = lax.bitwise_and(iota + dd, FUSE_DIM - 1)
                hv = plsc.load_gather(h_v, [cols, rows])
                tv = plsc.load_gather(t_v, [cols, rows])
                rv = plsc.load_gather(r_v, [rows, cols])
                return acc + hv * rv * tv

            acc = lax.fori_loop(0, FUSE_DIM, d_body,
                                jnp.zeros((16,), jnp.float32), unroll=8)
            out_v[pl.ds(sub * _SUB + g * 16, 16)] = acc
            return carry

        lax.fori_loop(0, _GROUPS, group_body, 0)

    cps = start(0)
    for sub in range(_NSUB):
        for cp in cps:
            cp.wait()
        if sub + 1 < _NSUB:
            cps = start(sub + 1)
        compute(sub)

    pltpu.sync_copy(out_v, out_hbm.at[pl.ds(base, _CHUNK)])


@jax.jit
def kernel(h_emb, r_idx, t_emb, rel_emb):
    ht = h_emb.T
    tt = t_emb.T
    relp = jnp.pad(rel_emb, ((0, 0), (0, 128 - FUSE_DIM)))
    mesh = plsc.VectorSubcoreMesh(core_axis_name="c", subcore_axis_name="s")
    buf = [
        pltpu.VMEM((_SUB,), jnp.int32),
        pltpu.VMEM((FUSE_DIM, _SUB), jnp.float32),
        pltpu.VMEM((FUSE_DIM, _SUB), jnp.float32),
        pltpu.VMEM((_SUB, 128), jnp.float32),
        pltpu.SemaphoreType.DMA,
        pltpu.SemaphoreType.DMA,
        pltpu.SemaphoreType.DMA,
    ]
    run = functools.partial(
        pl.kernel,
        mesh=mesh,
        compiler_params=pltpu.CompilerParams(needs_layout_passes=False),
        out_type=jax.ShapeDtypeStruct((BATCH,), jnp.float32),
        scratch_types=buf + buf + [pltpu.VMEM((_CHUNK,), jnp.float32)],
    )(_sc_kernel)
    return run(ht, r_idx.astype(jnp.int32), tt, relp)


# speedup vs baseline: 2.0488x; 1.0937x over previous
import functools

import jax
import jax.numpy as jnp
from jax import lax
from jax.experimental import pallas as pl
from jax.experimental.pallas import tpu as pltpu
from jax.experimental.pallas import tpu_sc as plsc

NUM_RELATIONS = 1000
FUSE_DIM = 64
BATCH = 16384

_NC = 2
_NS = 16
_NW = _NC * _NS
_CHUNK = BATCH // _NW
_SUB = 128
_NSUB = _CHUNK // _SUB
_GROUPS = _SUB // 16


def _sc_kernel(ht_hbm, idx_hbm, tt_hbm, rel_hbm, out_hbm,
               idx_v0, h_v0, t_v0, r_v0, sh0, st0, sr0,
               idx_v1, h_v1, t_v1, r_v1, sh1, st1, sr1,
               out_v, rel_sh):
    bufs = ((idx_v0, h_v0, t_v0, r_v0, sh0, st0, sr0),
            (idx_v1, h_v1, t_v1, r_v1, sh1, st1, sr1))
    wid = lax.axis_index("s") * _NC + lax.axis_index("c")
    base = wid * _CHUNK

    iota = lax.iota(jnp.int32, 16)

    @pl.when(lax.axis_index("s") == 0)
    def _():
        pltpu.sync_copy(rel_hbm, rel_sh)

    plsc.subcore_barrier()

    def start(sub):
        idx_v, h_v, t_v, r_v, sh, st, sr = bufs[sub % 2]
        sbase = pl.multiple_of(base + sub * _SUB, _SUB)
        pltpu.sync_copy(idx_hbm.at[pl.ds(sbase, _SUB)], idx_v)
        return (pltpu.async_copy(rel_sh.at[idx_v], r_v, sr),
                pltpu.async_copy(ht_hbm.at[:, pl.ds(sbase, _SUB)], h_v, sh),
                pltpu.async_copy(tt_hbm.at[:, pl.ds(sbase, _SUB)], t_v, st))

    def compute(sub):
        _, h_v, t_v, r_v, *_ = bufs[sub % 2]

        def group_body(g, carry):
            rows = iota + g * 16

            def d_body(dd, acc):
                cols = lax.bitwise_and(iota + dd, FUSE_DIM - 1)
                hv = plsc.load_gather(h_v, [cols, rows])
                tv = plsc.load_gather(t_v, [cols, rows])
                rv = plsc.load_gather(r_v, [rows, cols])
                return acc + hv * rv * tv

            acc = lax.fori_loop(0, FUSE_DIM, d_body,
                                jnp.zeros((16,), jnp.float32), unroll=8)
            out_v[pl.ds(sub * _SUB + g * 16, 16)] = acc
            return carry

        lax.fori_loop(0, _GROUPS, group_body, 0)

    cps = start(0)
    for sub in range(_NSUB):
        for cp in cps:
            cp.wait()
        if sub + 1 < _NSUB:
            cps = start(sub + 1)
        compute(sub)

    pltpu.sync_copy(out_v, out_hbm.at[pl.ds(base, _CHUNK)])


@jax.jit
def kernel(h_emb, r_idx, t_emb, rel_emb):
    ht = h_emb.T
    tt = t_emb.T
    relp = jnp.pad(rel_emb, ((0, 0), (0, 128 - FUSE_DIM)))
    mesh = plsc.VectorSubcoreMesh(core_axis_name="c", subcore_axis_name="s")
    buf = [
        pltpu.VMEM((_SUB,), jnp.int32),
        pltpu.VMEM((FUSE_DIM, _SUB), jnp.float32),
        pltpu.VMEM((FUSE_DIM, _SUB), jnp.float32),
        pltpu.VMEM((_SUB, 128), jnp.float32),
        pltpu.SemaphoreType.DMA,
        pltpu.SemaphoreType.DMA,
        pltpu.SemaphoreType.DMA,
    ]
    run = functools.partial(
        pl.kernel,
        mesh=mesh,
        compiler_params=pltpu.CompilerParams(needs_layout_passes=False),
        out_type=jax.ShapeDtypeStruct((BATCH,), jnp.float32),
        scratch_types=buf + buf + [
            pltpu.VMEM((_CHUNK,), jnp.float32),
            pltpu.VMEM_SHARED((NUM_RELATIONS, 128), jnp.float32),
        ],
    )(_sc_kernel)
    return run(ht, r_idx.astype(jnp.int32), tt, relp)
